# Initial kernel scaffold; baseline (speedup 1.0000x reference)
import jax
import jax.numpy as jnp
from jax.experimental import pallas as pl

_NW = 7
_HEADS = 4
_TOPK = 4
_KVW = 4


def _dw(x, w, b, k):
    out = jax.lax.conv_general_dilated(x, w, window_strides=(1, 1),
        padding=[(k // 2, k // 2)] * 2,
        dimension_numbers=('NCHW', 'OIHW', 'NCHW'),
        feature_group_count=x.shape[1])
    return out + b[None, :, None, None]


def _gs(x, grid):
    n, c, Hh, Ww = x.shape
    gx = (grid[..., 0] + 1.0) * (Ww - 1) / 2.0
    gy = (grid[..., 1] + 1.0) * (Hh - 1) / 2.0
    x0 = jnp.floor(gx); y0 = jnp.floor(gy)
    xf = x.reshape(n, c, Hh * Ww)
    def gather(xi, yi):
        valid = (xi >= 0) & (xi <= Ww - 1) & (yi >= 0) & (yi <= Hh - 1)
        xi_c = jnp.clip(xi, 0, Ww - 1).astype(jnp.int32)
        yi_c = jnp.clip(yi, 0, Hh - 1).astype(jnp.int32)
        idx = (yi_c * Ww + xi_c).reshape(n, 1, -1)
        g = jnp.take_along_axis(xf, jnp.broadcast_to(idx, (n, c, idx.shape[-1])), axis=2)
        return g * valid.reshape(n, 1, -1).astype(x.dtype)
    Ho, Wo = grid.shape[1], grid.shape[2]
    g00 = gather(x0, y0); g01 = gather(x0 + 1.0, y0)
    g10 = gather(x0, y0 + 1.0); g11 = gather(x0 + 1.0, y0 + 1.0)
    wx1 = (gx - x0).reshape(n, 1, -1); wx0 = 1.0 - wx1
    wy1 = (gy - y0).reshape(n, 1, -1); wy0 = 1.0 - wy1
    out = g00 * wx0 * wy0 + g01 * wx1 * wy0 + g10 * wx0 * wy1 + g11 * wx1 * wy1
    return out.reshape(n, c, Ho, Wo)


def kernel(x, Wq, bq, Wkv, bkv, sec_w, sec_b, off_dw_w, off_dw_b, bn_g, bn_b, off_pw_w):
    n, c, Hh, Ww = x.shape
    NW, HEADS, TOPK, KVW = _NW, _HEADS, _TOPK, _KVW
    SCALE = c ** -0.5
    hw = Hh // NW; ww = Ww // NW; p2 = NW * NW
    q = jnp.einsum('nchw,oc->nohw', x, Wq) + bq[None, :, None, None]
    off = _dw(q, off_dw_w, off_dw_b, 9)
    off = off / jnp.sqrt(1.0 + 1e-5) * bn_g[None, :, None, None] + bn_b[None, :, None, None]
    off = jnp.einsum('nchw,oc->nohw', off, off_pw_w)
    off = off.transpose(0, 2, 3, 1)
    ry, rx = jnp.meshgrid(jnp.linspace(0.5, Hh - 0.5, Hh), jnp.linspace(0.5, Ww - 0.5, Ww), indexing='ij')
    ref = jnp.stack([ry / (Hh - 1.0) * 2.0 - 1.0, rx / (Ww - 1.0) * 2.0 - 1.0], axis=-1)
    pos = jnp.clip(off + jax.lax.stop_gradient(ref[None]), -0.12, 0.12)
    x_sampled = _gs(x, pos[..., ::-1])
    kv = jnp.einsum('nchw,oc->nohw', x_sampled, Wkv) + bkv[None, :, None, None]
    query = q.reshape(n, c, NW, hw, NW, ww).transpose(0, 2, 4, 3, 5, 1).reshape(n, p2, hw, ww, c)
    kvw = kv.reshape(n, 2 * c, NW, hw, NW, ww).transpose(0, 2, 4, 3, 5, 1).reshape(n, p2, hw, ww, 2 * c)
    query_pixel = query.reshape(n, p2, hw * ww, c)
    kh = hw // KVW; kw = ww // KVW
    kv_pix = kvw.transpose(0, 1, 4, 2, 3).reshape(n * p2, 2 * c, KVW, kh, KVW, kw).max(axis=(3, 5))
    kv_pix = kv_pix.reshape(n, p2, 2 * c, KVW * KVW).transpose(0, 1, 3, 2)
    q_win = query.mean(axis=(2, 3))
    k_win = kvw[..., :c].mean(axis=(2, 3))
    logit = jnp.einsum('npc,nqc->npq', jax.lax.stop_gradient(q_win) * SCALE, jax.lax.stop_gradient(k_win))
    topv, topi = jax.lax.top_k(logit, TOPK)
    kv_sel = jnp.take_along_axis(kv_pix[:, None], topi[:, :, :, None, None], axis=2)
    k_sel = kv_sel[..., :c]; v_sel = kv_sel[..., c:]
    hd = c // HEADS; L = TOPK * KVW * KVW
    k_mh = k_sel.reshape(n * p2, L, HEADS, hd).transpose(0, 2, 3, 1)
    v_mh = v_sel.reshape(n * p2, L, HEADS, hd).transpose(0, 2, 1, 3)
    q_mh = query_pixel.reshape(n * p2, hw * ww, HEADS, hd).transpose(0, 2, 1, 3)
    attn = jax.nn.softmax(jnp.einsum('bmqd,bmdl->bmql', q_mh * SCALE, k_mh), axis=-1)
    out = jnp.einsum('bmql,bmld->bmqd', attn, v_mh)
    out = out.reshape(n, NW, NW, HEADS, hw, ww, hd).transpose(0, 1, 4, 2, 5, 3, 6).reshape(n, Hh, Ww, c)
    v_full = kvw[..., c:].reshape(n, NW, NW, hw, ww, c).transpose(0, 5, 1, 3, 2, 4).reshape(n, c, Hh, Ww)
    lepe = _dw(v_full, sec_w, sec_b, 3).transpose(0, 2, 3, 1)
    out = out + lepe
    return out.transpose(0, 3, 1, 2)


# speedup vs baseline: 15.5434x; 15.5434x over previous
import functools

import jax
import jax.numpy as jnp
from jax import lax
from jax.experimental import pallas as pl
from jax.experimental.pallas import tpu as pltpu
from jax.experimental.pallas import tpu_sc as plsc

F32 = jnp.float32
BF16 = jnp.bfloat16

B = 2
C = 384
H = 224
NWIN = 7
WS = 32
P2 = NWIN * NWIN
HEADS = 4
HD = C // HEADS
TOPK = 4
KVW = 4
L = TOPK * KVW * KVW
PATCH0 = 98
PATCHN = 28
NPIX = H * H
SCALE = C ** -0.5


def _bf(a):
    return a.astype(BF16)


K1R = 16


def _k1_body(x_ref, wq_ref, bq_ref, q_ref, qsum_ref):
    h2 = pl.program_id(1)
    xb = _bf(x_ref[0].reshape(C, K1R * H))
    wqb = _bf(wq_ref[...])
    qt = jax.lax.dot_general(wqb, xb, (((1,), (0,)), ((), ())),
                             preferred_element_type=F32)
    qt = qt + bq_ref[...].reshape(C, 1)
    q_ref[0] = _bf(qt).reshape(C, K1R, H)
    s = qt.reshape(C, K1R, NWIN, WS).sum(axis=(1, 3))

    @pl.when(h2 % 2 == 0)
    def _():
        qsum_ref[0, 0] = s

    @pl.when(h2 % 2 == 1)
    def _():
        qsum_ref[0, 0] = qsum_ref[0, 0] + s


def _k1(x, Wq, bq):
    grid = (B, H // K1R)
    return pl.pallas_call(
        _k1_body,
        grid=grid,
        in_specs=[
            pl.BlockSpec((1, C, K1R, H), lambda b, r: (b, 0, r, 0)),
            pl.BlockSpec((C, C), lambda b, r: (0, 0)),
            pl.BlockSpec((1, C), lambda b, r: (0, 0)),
        ],
        out_specs=[
            pl.BlockSpec((1, C, K1R, H), lambda b, r: (b, 0, r, 0)),
            pl.BlockSpec((1, 1, C, NWIN), lambda b, r: (b, r // 2, 0, 0)),
        ],
        out_shape=[
            jax.ShapeDtypeStruct((B, C, H, H), BF16),
            jax.ShapeDtypeStruct((B, NWIN, C, NWIN), F32),
        ],
    )(x, Wq, bq.reshape(1, C))


CCH = 32
NCC = C // CCH


def _k2_body(q_ref, dww_ref, dwb_ref, bng_ref, bnb_ref, pww_ref,
             refy_ref, refx_ref, off2_ref, idx_ref, wts_ref, qpad_ref,
             acc_ref):
    cc = pl.program_id(1)
    b = pl.program_id(0)
    qpad_ref[...] = jnp.pad(q_ref[0], ((0, 0), (4, 4), (4, 4)))
    acc_ref[...] = jnp.zeros((CCH, H, H), F32)
    for dy in range(9):
        for dx in range(9):
            sl = qpad_ref[:, dy:dy + H, dx:dx + H]
            wv = dww_ref[dy, :, dx].astype(F32)
            acc_ref[...] = acc_ref[...] + sl.astype(F32) * wv[:, None, None]
    acc = acc_ref[...] + dwb_ref[...].reshape(CCH, 1, 1)
    acc = acc / jnp.sqrt(F32(1.0 + 1e-5)) * bng_ref[...].reshape(CCH, 1, 1) \
        + bnb_ref[...].reshape(CCH, 1, 1)
    sb = _bf(acc).astype(F32)
    pwb = _bf(pww_ref[0]).astype(F32)
    part = jnp.stack([
        (sb * pwb[0][:, None, None]).sum(axis=0),
        (sb * pwb[1][:, None, None]).sum(axis=0),
    ])

    @pl.when(cc == 0)
    def _():
        off2_ref[0] = part

    @pl.when(cc > 0)
    def _():
        off2_ref[0] = off2_ref[0] + part

    @pl.when(cc == NCC - 1)
    def _():
        off2 = off2_ref[0]
        pos_y = jnp.clip(off2[0] + refy_ref[...], -0.12, 0.12)
        pos_x = jnp.clip(off2[1] + refx_ref[...], -0.12, 0.12)
        gx = (pos_x + 1.0) * F32(H - 1) / 2.0
        gy = (pos_y + 1.0) * F32(H - 1) / 2.0
        x0 = jnp.floor(gx)
        y0 = jnp.floor(gy)
        wx1 = gx - x0
        wy1 = gy - y0
        x0i = x0.astype(jnp.int32) - PATCH0
        y0i = y0.astype(jnp.int32) - PATCH0
        idx_ref[0, 0] = b * (PATCHN * PATCHN) + y0i * PATCHN + x0i
        wts_ref[0, 0] = 1.0 - wx1
        wts_ref[0, 1] = wx1
        wts_ref[0, 2] = 1.0 - wy1
        wts_ref[0, 3] = wy1


def _k2(q, off_dw_w, off_dw_b, bn_g, bn_b, off_pw_w, refy, refx):
    grid = (B, NCC)
    return pl.pallas_call(
        _k2_body,
        grid=grid,
        in_specs=[
            pl.BlockSpec((1, CCH, H, H), lambda b, cc: (b, cc, 0, 0)),
            pl.BlockSpec((9, CCH, 9), lambda b, cc: (0, cc, 0)),
            pl.BlockSpec((1, 1, CCH), lambda b, cc: (cc, 0, 0)),
            pl.BlockSpec((1, 1, CCH), lambda b, cc: (cc, 0, 0)),
            pl.BlockSpec((1, 1, CCH), lambda b, cc: (cc, 0, 0)),
            pl.BlockSpec((1, 2, CCH), lambda b, cc: (cc, 0, 0)),
            pl.BlockSpec((H, H), lambda b, cc: (0, 0)),
            pl.BlockSpec((H, H), lambda b, cc: (0, 0)),
        ],
        out_specs=[
            pl.BlockSpec((1, 2, H, H), lambda b, cc: (b, 0, 0, 0)),
            pl.BlockSpec((1, 1, H, H), lambda b, cc: (b, 0, 0, 0)),
            pl.BlockSpec((1, 4, H, H), lambda b, cc: (b, 0, 0, 0)),
        ],
        out_shape=[
            jax.ShapeDtypeStruct((B, 2, H, H), F32),
            jax.ShapeDtypeStruct((B, 1, H, H), jnp.int32),
            jax.ShapeDtypeStruct((B, 4, H, H), F32),
        ],
        scratch_shapes=[pltpu.VMEM((CCH, H + 8, H + 8), BF16),
                        pltpu.VMEM((CCH, H, H), F32)],
    )(q, _bf(off_dw_w[:, 0]).transpose(1, 0, 2), off_dw_b.reshape(NCC, 1, CCH),
      bn_g.reshape(NCC, 1, CCH), bn_b.reshape(NCC, 1, CCH),
      off_pw_w.reshape(2, NCC, CCH).transpose(1, 0, 2), refy, refx)


SC_CHUNK = 32
NWORK = 32
TOT = B * NPIX
PPW = TOT // NWORK
NG = (4 * C) // 16


def _k3_body(t4_hbm, idx_hbm, w_hbm, out_hbm, idx_v, w_v, rows_v, out_v, sem):
    wid = lax.axis_index("s") * 2 + lax.axis_index("c")
    base = wid * PPW

    @pl.loop(0, PPW // SC_CHUNK)
    def _(g):
        off = base + g * SC_CHUNK
        pltpu.sync_copy(idx_hbm.at[pl.ds(off, SC_CHUNK)], idx_v)
        pltpu.sync_copy(w_hbm.at[pl.ds(off, SC_CHUNK)], w_v)
        pltpu.async_copy(t4_hbm.at[idx_v], rows_v, sem).wait()

        @pl.loop(0, SC_CHUNK)
        def _(p):
            wx0 = w_v[p, pl.ds(0, 16)]
            wx1 = w_v[p, pl.ds(16, 16)]
            wy0 = w_v[p, pl.ds(32, 16)]
            wy1 = w_v[p, pl.ds(48, 16)]
            for gi in range(C // 16):
                g00 = rows_v[p, pl.ds(gi * 16, 16)]
                g01 = rows_v[p, pl.ds(C + gi * 16, 16)]
                g10 = rows_v[p, pl.ds(2 * C + gi * 16, 16)]
                g11 = rows_v[p, pl.ds(3 * C + gi * 16, 16)]
                s = (g00 * wx0) * wy0 + (g01 * wx1) * wy0
                s = s + (g10 * wx0) * wy1
                s = s + (g11 * wx1) * wy1
                out_v[p, pl.ds(gi * 16, 16)] = s

        pltpu.sync_copy(out_v, out_hbm.at[pl.ds(off, SC_CHUNK)])


def _k3(table4, idx_flat, w_flat):
    mesh = plsc.VectorSubcoreMesh(core_axis_name="c", subcore_axis_name="s")
    k = pl.kernel(
        _k3_body,
        out_type=jax.ShapeDtypeStruct((TOT, C), F32),
        mesh=mesh,
        scratch_types=[
            pltpu.VMEM((SC_CHUNK,), jnp.int32),
            pltpu.VMEM((SC_CHUNK, 64), F32),
            pltpu.VMEM((SC_CHUNK, 4 * C), F32),
            pltpu.VMEM((SC_CHUNK, C), F32),
            pltpu.SemaphoreType.DMA,
        ],
    )
    return k(table4, idx_flat, w_flat)


def _k4_body(xs_ref, wkv_ref, bkv_ref, kvp_ref, kwin_ref, v_ref):
    wkvb = _bf(wkv_ref[...])
    xst = xs_ref[0].reshape(WS, NWIN, WS, C)
    for w2 in range(NWIN):
        xsb = _bf(xst[:, w2].reshape(WS * WS, C))
        kvt = jax.lax.dot_general(xsb, wkvb, (((1,), (1,)), ((), ())),
                                  preferred_element_type=F32)
        kvt = kvt + bkv_ref[...]
        kwin_ref[0, w2] = jnp.mean(kvt[:, :C], axis=0).reshape(1, C)
        pooled = kvt.reshape(KVW, 8, KVW, 8, 2 * C).max(axis=(1, 3))
        kvp_ref[0, w2] = _bf(pooled.reshape(KVW * KVW, 2 * C))
        v_ref[0, :, :, w2 * WS:(w2 + 1) * WS] = _bf(kvt[:, C:]).T.reshape(C, WS, WS)


def _k4(xs, Wkv, bkv):
    grid = (B, NWIN)
    return pl.pallas_call(
        _k4_body,
        grid=grid,
        in_specs=[
            pl.BlockSpec((1, WS, H, C), lambda b, r: (b, r, 0, 0)),
            pl.BlockSpec((2 * C, C), lambda b, r: (0, 0)),
            pl.BlockSpec((1, 2 * C), lambda b, r: (0, 0)),
        ],
        out_specs=[
            pl.BlockSpec((1, NWIN, KVW * KVW, 2 * C), lambda b, r: (b, r, 0, 0)),
            pl.BlockSpec((1, NWIN, 1, C), lambda b, r: (b, r, 0, 0)),
            pl.BlockSpec((1, C, WS, H), lambda b, r: (b, 0, r, 0)),
        ],
        out_shape=[
            jax.ShapeDtypeStruct((B, P2, KVW * KVW, 2 * C), BF16),
            jax.ShapeDtypeStruct((B, P2, 1, C), F32),
            jax.ShapeDtypeStruct((B, C, H, H), BF16),
        ],
    )(xs, Wkv, bkv.reshape(1, 2 * C))


def _k5_body(qwin_ref, kwin_ref, topi_ref):
    a = _bf(qwin_ref[0] * F32(SCALE))
    kb = _bf(kwin_ref[0])
    logit = jax.lax.dot_general(a, kb, (((1,), (1,)), ((), ())),
                                preferred_element_type=F32)
    cols = jax.lax.broadcasted_iota(jnp.int32, (P2, P2), 1)
    neg = jnp.full((P2, P2), -jnp.inf, F32)
    picks = []
    for k in range(TOPK):
        im = jnp.argmax(logit, axis=1).astype(jnp.int32)
        picks.append(im.reshape(P2, 1))
        logit = jnp.where(cols == im[:, None], neg, logit)
    picks.append(jnp.zeros((P2, 4), jnp.int32))
    topi_ref[0] = jnp.concatenate(picks, axis=1)


def _k5(qwin, kwin):
    return pl.pallas_call(
        _k5_body,
        grid=(B,),
        in_specs=[
            pl.BlockSpec((1, P2, C), lambda b: (b, 0, 0)),
            pl.BlockSpec((1, P2, C), lambda b: (b, 0, 0)),
        ],
        out_specs=pl.BlockSpec((1, P2, 8), lambda b: (b, 0, 0)),
        out_shape=jax.ShapeDtypeStruct((B, P2, 8), jnp.int32),
    )(qwin, kwin)


def _k6_body(topi_ref, q_ref, kvp_ref, out_ref):
    b = pl.program_id(0)
    w = pl.program_id(1)
    qs = _bf(q_ref[0, 0].astype(F32) * F32(SCALE))
    parts = []
    for k in range(TOPK):
        ti = topi_ref[b, w, k]
        parts.append(kvp_ref[0, pl.ds(ti * (KVW * KVW), KVW * KVW), :])
    ksel = jnp.concatenate(parts, axis=0)
    for h in range(HEADS):
        qh = qs[h * HD:(h + 1) * HD, :]
        kh = ksel[:, h * HD:(h + 1) * HD]
        lg = jax.lax.dot_general(kh, qh, (((1,), (0,)), ((), ())),
                                 preferred_element_type=F32)
        m = jnp.max(lg, axis=0, keepdims=True)
        e = jnp.exp(lg - m)
        p = e / jnp.sum(e, axis=0, keepdims=True)
        pb = _bf(p)
        vh = ksel[:, C + h * HD:C + (h + 1) * HD]
        oh = jax.lax.dot_general(vh, pb, (((0,), (0,)), ((), ())),
                                 preferred_element_type=F32)
        out_ref[0, 0, h * HD:(h + 1) * HD, :] = oh


def _k6(topi, q_flat, kvp_flat):
    grid = (B, P2)
    return pl.pallas_call(
        _k6_body,
        grid=grid,
        in_specs=[
            pl.BlockSpec(memory_space=pltpu.SMEM),
            pl.BlockSpec((1, 1, C, WS * WS), lambda b, w: (b, w, 0, 0)),
            pl.BlockSpec((1, P2 * KVW * KVW, 2 * C), lambda b, w: (b, 0, 0)),
        ],
        out_specs=pl.BlockSpec((1, 1, C, WS * WS), lambda b, w: (b, w, 0, 0)),
        out_shape=jax.ShapeDtypeStruct((B, P2, C, WS * WS), F32),
    )(topi, q_flat, kvp_flat)


LCH = 16
NLC = C // LCH


def _k7_body(v_ref, attn_ref, sw_ref, sb_ref, out_ref, vpad_ref):
    vpad_ref[...] = jnp.pad(v_ref[0], ((0, 0), (1, 1), (1, 1)))
    acc = jnp.zeros((LCH, H, H), F32)
    for dx in range(3):
        for dy in range(3):
            sl = vpad_ref[:, dy:dy + H, dx:dx + H]
            wv = sw_ref[dy, :, dx].astype(F32)
            acc = acc + sl.astype(F32) * wv[:, None, None]
    out_ref[0] = acc + sb_ref[...].reshape(LCH, 1, 1) + attn_ref[0]


def _k7(v, attn, sec_w, sec_b):
    grid = (B, NLC)
    return pl.pallas_call(
        _k7_body,
        grid=grid,
        in_specs=[
            pl.BlockSpec((1, LCH, H, H), lambda b, cc: (b, cc, 0, 0)),
            pl.BlockSpec((1, LCH, H, H), lambda b, cc: (b, cc, 0, 0)),
            pl.BlockSpec((3, LCH, 3), lambda b, cc: (0, cc, 0)),
            pl.BlockSpec((1, 1, LCH), lambda b, cc: (cc, 0, 0)),
        ],
        out_specs=pl.BlockSpec((1, LCH, H, H), lambda b, cc: (b, cc, 0, 0)),
        out_shape=jax.ShapeDtypeStruct((B, C, H, H), F32),
        scratch_shapes=[pltpu.VMEM((LCH, H + 2, H + 2), BF16)],
    )(v, attn, _bf(sec_w[:, 0]).transpose(1, 0, 2), sec_b.reshape(NLC, 1, LCH))


def kernel(x, Wq, bq, Wkv, bkv, sec_w, sec_b, off_dw_w, off_dw_b, bn_g, bn_b,
           off_pw_w):
    ry, rx = jnp.meshgrid(jnp.linspace(0.5, H - 0.5, H),
                          jnp.linspace(0.5, H - 0.5, H), indexing='ij')
    refy = (ry / (H - 1.0) * 2.0 - 1.0).astype(F32)
    refx = (rx / (H - 1.0) * 2.0 - 1.0).astype(F32)

    q, qsum = _k1(x, Wq, bq)
    qwin = (qsum.transpose(0, 1, 3, 2).reshape(B, P2, C)) / F32(WS * WS)

    off2, idx, wts = _k2(q, off_dw_w, off_dw_b, bn_g, bn_b, off_pw_w,
                         refy, refx)

    patch = x[:, :, PATCH0:PATCH0 + PATCHN, PATCH0:PATCH0 + PATCHN]
    pf = patch.transpose(0, 2, 3, 1).reshape(B, PATCHN * PATCHN, C)
    pfp = jnp.pad(pf, ((0, 0), (0, PATCHN + 1), (0, 0)))
    nn = PATCHN * PATCHN
    table4 = jnp.concatenate(
        [pfp[:, 0:nn], pfp[:, 1:nn + 1],
         pfp[:, PATCHN:nn + PATCHN], pfp[:, PATCHN + 1:nn + PATCHN + 1]],
        axis=-1).reshape(B * nn, 4 * C)

    idx_flat = idx.reshape(TOT)
    w_flat = wts.transpose(0, 2, 3, 1).reshape(TOT, 4)
    w_rep = jnp.broadcast_to(w_flat[:, :, None], (TOT, 4, 16)).reshape(TOT, 64)
    xs = _k3(table4, idx_flat, w_rep).reshape(B, H, H, C)

    kvp, kwin, v = _k4(xs, Wkv, bkv)
    topi = _k5(qwin, kwin.reshape(B, P2, C))
    q_flat = (q.reshape(B, C, NWIN, WS, NWIN, WS)
              .transpose(0, 2, 4, 1, 3, 5).reshape(B, P2, C, WS * WS))
    attn_f = _k6(topi, q_flat, kvp.reshape(B, P2 * KVW * KVW, 2 * C))
    attn = (attn_f.reshape(B, NWIN, NWIN, C, WS, WS)
            .transpose(0, 3, 1, 4, 2, 5).reshape(B, C, H, H))
    return _k7(v, attn, sec_w, sec_b)
